# Initial kernel scaffold; baseline (speedup 1.0000x reference)
import jax
import jax.numpy as jnp
from jax.experimental import pallas as pl


def kernel(x, W_enc, b_enc, W_dec, b_dec):
    raise NotImplementedError("write your pallas kernel here")


# speedup vs baseline: 15.8923x; 15.8923x over previous
import jax
import jax.numpy as jnp
import numpy as np
from jax.experimental import pallas as pl

D_IN = 1024
D_SAE = 4096
T = 8
K = 32
RB = 256

_SIGN = int(np.int32(np.uint32(0x80000000)))


def _encode_kernel(x_ref, We_ref, be_ref, bd_ref, z_ref):
    xc = x_ref[...] - bd_ref[0]
    pre = jax.lax.dot_general(
        xc, We_ref[0],
        (((1,), (0,)), ((), ())),
        preferred_element_type=jnp.float32,
    ) + be_ref[0]

    u = jax.lax.bitcast_convert_type(pre, jnp.int32)
    s = u ^ ((u >> 31) & jnp.int32(0x7FFFFFFF))

    prefix = jnp.zeros((x_ref.shape[0], 1), jnp.int32)
    for i in range(31, -1, -1):
        bit = int(np.int32(np.uint32(1 << i)))
        cand = prefix | bit
        cnt = jnp.sum((s >= (cand ^ _SIGN)).astype(jnp.int32), axis=1,
                      keepdims=True)
        prefix = jnp.where(cnt >= K, cand, prefix)
    tau = prefix ^ _SIGN

    mask = (s >= tau) & (pre > 0.0)
    z_ref[...] = jnp.where(mask, pre, 0.0)


def _decode_kernel(z_ref, x_ref, Wd_ref, bd_ref, xhat_ref, loss_ref):
    t = pl.program_id(0)
    rb = pl.program_id(1)
    xh = jax.lax.dot_general(
        z_ref[...], Wd_ref[0],
        (((1,), (1,)), ((), ())),
        preferred_element_type=jnp.float32,
    ) + bd_ref[0]
    xhat_ref[...] = xh
    err = xh - x_ref[...]

    @pl.when((t == 0) & (rb == 0))
    def _():
        loss_ref[...] = jnp.zeros((1, 1), jnp.float32)

    loss_ref[...] += jnp.sum(err * err).reshape(1, 1)


def kernel(x, W_enc, b_enc, W_dec, b_dec):
    B = x.shape[0]
    nb = B // RB
    grid = (T, nb)

    x2 = x.reshape(B, T * D_IN)
    be = b_enc.reshape(T, 1, D_SAE)
    bd = b_dec.reshape(T, 1, D_IN)

    z2 = pl.pallas_call(
        _encode_kernel,
        grid=grid,
        in_specs=[
            pl.BlockSpec((RB, D_IN), lambda t, rb: (rb, t)),
            pl.BlockSpec((1, D_IN, D_SAE), lambda t, rb: (t, 0, 0)),
            pl.BlockSpec((1, 1, D_SAE), lambda t, rb: (t, 0, 0)),
            pl.BlockSpec((1, 1, D_IN), lambda t, rb: (t, 0, 0)),
        ],
        out_specs=pl.BlockSpec((RB, D_SAE), lambda t, rb: (rb, t)),
        out_shape=jax.ShapeDtypeStruct((B, T * D_SAE), jnp.float32),
    )(x2, W_enc, be, bd)

    xhat2, loss_sum = pl.pallas_call(
        _decode_kernel,
        grid=grid,
        in_specs=[
            pl.BlockSpec((RB, D_SAE), lambda t, rb: (rb, t)),
            pl.BlockSpec((RB, D_IN), lambda t, rb: (rb, t)),
            pl.BlockSpec((1, D_IN, D_SAE), lambda t, rb: (t, 0, 0)),
            pl.BlockSpec((1, 1, D_IN), lambda t, rb: (t, 0, 0)),
        ],
        out_specs=[
            pl.BlockSpec((RB, D_IN), lambda t, rb: (rb, t)),
            pl.BlockSpec((1, 1), lambda t, rb: (0, 0)),
        ],
        out_shape=[
            jax.ShapeDtypeStruct((B, T * D_IN), jnp.float32),
            jax.ShapeDtypeStruct((1, 1), jnp.float32),
        ],
    )(z2, x2, W_dec, bd)

    loss = loss_sum[0, 0] / jnp.float32(B * T * D_IN)
    return (loss, xhat2.reshape(B, T, D_IN), z2.reshape(B, T, D_SAE))
